# baseline (speedup 1.0000x reference)
import functools

import jax
import jax.numpy as jnp
from jax import lax
from jax.experimental import pallas as pl
from jax.experimental.pallas import tpu as pltpu
from jax.experimental.pallas import tpu_sc as plsc

_B = 4
_N = 4096
_DEG = 16
_D = 64
_ANN = 32
_NT = 32
_NET = 16
_STEPS = 2

_NP = _N // 2
_PD = 2 * _D

_NC = 2
_NS = 16
_LANES = 128
_E = _B * _N * _DEG
_EROWS = _E // _LANES
_CHUNKS = 2 * _E // (_NC * _NS * _LANES)
_ACC_ROWS = 2 * 2 * _N
_WB = _ACC_ROWS // _NS


def _blockdiag(m):
    z = jnp.zeros((_D, _D), jnp.float32)
    return jnp.concatenate([jnp.concatenate([m, z], axis=1),
                            jnp.concatenate([z, m], axis=1)], axis=0)


def _index_body(et_ref, nb_ref, gidx_ref, didx_ref):
    et = et_ref[...]
    nb = nb_ref[...]
    f = (lax.broadcasted_iota(jnp.int32, (_EROWS, _LANES), 0) * _LANES
         + lax.broadcasted_iota(jnp.int32, (_EROWS, _LANES), 1))
    b = f // (_N * _DEG)
    n = (f // _DEG) % _N
    bloc = (b % 2) * _N
    g_out = (et - 1) * (_B * _N) + b * _N + nb
    g_in = (et + _NET // 2 - 1) * (_B * _N) + b * _N + n
    d_out = bloc + n
    d_in = 2 * _N + bloc + nb
    half = _EROWS // _NC
    for c in range(_NC):
        sl = slice(c * half, (c + 1) * half)
        gidx_ref[c, 0:half] = g_out[sl]
        gidx_ref[c, half:2 * half] = g_in[sl]
        didx_ref[c, 0:half] = d_out[sl]
        didx_ref[c, half:2 * half] = d_in[sl]


def _edge_indices(A):
    et = A[..., 0].reshape(_EROWS, _LANES)
    nb = A[..., 1].reshape(_EROWS, _LANES)
    sh = jax.ShapeDtypeStruct((_NC, 2 * _EROWS // _NC, _LANES), jnp.int32)
    return pl.pallas_call(_index_body, out_shape=(sh, sh))(et, nb)


def _emit_transform(x, m3_ref, biasP_ref, tr_ref):
    for t in range(_NET):
        mb = _blockdiag(m3_ref[t])
        y = lax.dot_general(x, mb, (((1,), (1,)), ((), ())),
                            preferred_element_type=jnp.float32)
        tr_ref[t, 0] = (y + biasP_ref[t]).astype(jnp.bfloat16)


def _gru_value(aip, aop, stp, wr_ref, wz_ref, wh_ref, br_ref, bz_ref, bh_ref):
    def mm3(x0, x1, x2, w_ref):
        w = w_ref[...]
        acc = jnp.zeros((_NP, _PD), jnp.float32)
        for k, x in enumerate((x0, x1, x2)):
            wb = _blockdiag(w[k * _D:(k + 1) * _D])
            acc = acc + jnp.dot(x, wb, preferred_element_type=jnp.float32)
        return acc

    r = jax.nn.sigmoid(mm3(aip, aop, stp, wr_ref) + br_ref[...])
    z = jax.nn.sigmoid(mm3(aip, aop, stp, wz_ref) + bz_ref[...])
    hh = jnp.tanh(mm3(aip, aop, r * stp, wh_ref) + bh_ref[...])
    return (1.0 - z) * stp + z * hh


def _init_tr_body(ids_ref, te_ref, m3_ref, biasP_ref, st_ref, tr_ref):
    ids = ids_ref[0]
    te = te_ref[...]
    iota = lax.broadcasted_iota(jnp.int32, (_NP, _NT), 1)
    halves = []
    zpad = jnp.zeros((_NP, _D - _ANN), jnp.float32)
    for h in range(2):
        idh = ids[:, h:h + 1]
        oh = jnp.where((idh - 1 == iota) & (idh > 0), 1.0, 0.0)
        halves.append(jnp.dot(oh, te, preferred_element_type=jnp.float32))
        halves.append(zpad)
    x = jnp.concatenate(halves, axis=1)
    st_ref[0] = x
    _emit_transform(x, m3_ref, biasP_ref, tr_ref)


def _init_transform(annotation_id, typeEmbed, M3, biasP):
    return pl.pallas_call(
        _init_tr_body,
        grid=(_B,),
        in_specs=[
            pl.BlockSpec((1, _NP, 2), lambda b: (b, 0, 0)),
            pl.BlockSpec((_NT, _ANN), lambda b: (0, 0)),
            pl.BlockSpec((_NET, _D, _D), lambda b: (0, 0, 0)),
            pl.BlockSpec((_NET, 1, _PD), lambda b: (0, 0, 0)),
        ],
        out_specs=(
            pl.BlockSpec((1, _NP, _PD), lambda b: (b, 0, 0)),
            pl.BlockSpec((_NET, 1, _NP, _PD), lambda b: (0, b, 0, 0)),
        ),
        out_shape=(
            jax.ShapeDtypeStruct((_B, _NP, _PD), jnp.float32),
            jax.ShapeDtypeStruct((_NET, _B, _NP, _PD), jnp.bfloat16),
        ),
    )(annotation_id.reshape(_B, _NP, 2), typeEmbed, M3, biasP)


_NBUF = 4


@functools.cache
def _make_sc_edge_pass():
    mesh = plsc.VectorSubcoreMesh(core_axis_name="c", subcore_axis_name="s")

    @functools.partial(
        pl.kernel,
        out_type=jax.ShapeDtypeStruct((2 * _B * _N, _D), jnp.bfloat16),
        mesh=mesh,
        scratch_types=[
            pltpu.VMEM((_CHUNKS, _LANES), jnp.int32),
            pltpu.VMEM((_CHUNKS, _LANES), jnp.int32),
            pltpu.VMEM((_NBUF, _LANES, _D), jnp.bfloat16),
            pltpu.VMEM_SHARED((_ACC_ROWS, _D), jnp.bfloat16),
        ] + [pltpu.SemaphoreType.DMA] * _NBUF,
        compiler_params=pltpu.CompilerParams(use_tc_tiling_on_sc=False),
    )
    def _sc_edge_pass(tr_hbm, gidx_hbm, didx_hbm, out_hbm,
                      gidx_v, didx_v, rows_v, acc, *sems):
        c = lax.axis_index("c")
        s = lax.axis_index("s")
        pltpu.sync_copy(gidx_hbm.at[c, pl.ds(s * _CHUNKS, _CHUNKS)], gidx_v)
        pltpu.sync_copy(didx_hbm.at[c, pl.ds(s * _CHUNKS, _CHUNKS)], didx_v)

        def _zero_row(i, carry):
            for j in range(_D // 32):
                rows_v[0, i, pl.ds(j * 32, 32)] = jnp.zeros((32,), jnp.bfloat16)
            return carry

        lax.fori_loop(0, _LANES, _zero_row, 0)
        for k in range(_WB // _LANES):
            pltpu.sync_copy(rows_v.at[0],
                            acc.at[pl.ds(s * _WB + k * _LANES, _LANES)])
        plsc.subcore_barrier()

        def _gather_start(i, b):
            pltpu.async_copy(tr_hbm.at[gidx_v.at[i]], rows_v.at[b], sems[b])

        def _gather_wait(i, b):
            pltpu.make_async_copy(tr_hbm.at[gidx_v.at[i]], rows_v.at[b],
                                  sems[b]).wait()

        def _scatter(i, b):
            pltpu.sync_copy(rows_v.at[b], acc.at[didx_v.at[i]], add=True)

        for b in range(_NBUF):
            _gather_start(b, b)

        def _quad(k, carry):
            i = _NBUF * k
            for b in range(_NBUF):
                _gather_wait(i + b, b)
                _scatter(i + b, b)
                _gather_start(i + _NBUF + b, b)
            return carry

        lax.fori_loop(0, _CHUNKS // _NBUF - 1, _quad, 0)
        i = _CHUNKS - _NBUF
        for b in range(_NBUF):
            _gather_wait(i + b, b)
            _scatter(i + b, b)
        plsc.subcore_barrier()

        base = s * _WB + (s // (_NS // 2)) * (2 * _N) + c * (2 * _N)
        pltpu.sync_copy(acc.at[pl.ds(s * _WB, _WB)], out_hbm.at[pl.ds(base, _WB)])

    return _sc_edge_pass


def _gru_tr_body(ai_ref, ao_ref, st_ref, wr_ref, wz_ref, wh_ref,
                 br_ref, bz_ref, bh_ref, m3_ref, biasP_ref, out_ref, tr_ref):
    ns = _gru_value(ai_ref[0, 0], ao_ref[0, 0], st_ref[0],
                    wr_ref, wz_ref, wh_ref, br_ref, bz_ref, bh_ref)
    out_ref[0] = ns
    _emit_transform(ns, m3_ref, biasP_ref, tr_ref)


def _gru_transform(acat2, state3, W_r, W_z, W_h, brP, bzP, bhP, M3, biasP):
    aspec_in = pl.BlockSpec((1, 1, _NP, _PD), lambda b: (1, b, 0, 0))
    aspec_out = pl.BlockSpec((1, 1, _NP, _PD), lambda b: (0, b, 0, 0))
    full3 = pl.BlockSpec((1, _NP, _PD), lambda b: (b, 0, 0))
    wspec = pl.BlockSpec((3 * _D, _D), lambda b: (0, 0))
    bspec = pl.BlockSpec((1, _PD), lambda b: (0, 0))
    return pl.pallas_call(
        _gru_tr_body,
        grid=(_B,),
        in_specs=[aspec_in, aspec_out, full3, wspec, wspec, wspec,
                  bspec, bspec, bspec,
                  pl.BlockSpec((_NET, _D, _D), lambda b: (0, 0, 0)),
                  pl.BlockSpec((_NET, 1, _PD), lambda b: (0, 0, 0))],
        out_specs=(
            full3,
            pl.BlockSpec((_NET, 1, _NP, _PD), lambda b: (0, b, 0, 0)),
        ),
        out_shape=(
            jax.ShapeDtypeStruct((_B, _NP, _PD), jnp.float32),
            jax.ShapeDtypeStruct((_NET, _B, _NP, _PD), jnp.bfloat16),
        ),
    )(acat2, acat2, state3, W_r, W_z, W_h, brP, bzP, bhP, M3, biasP)


def _gru_read_body(ai_ref, ao_ref, st_ref, wr_ref, wz_ref, wh_ref,
                   br_ref, bz_ref, bh_ref, st0_ref, wa_ref, wo_ref,
                   ba_ref, bo_ref, out_ref):
    ns = _gru_value(ai_ref[0, 0], ao_ref[0, 0], st_ref[0],
                    wr_ref, wz_ref, wh_ref, br_ref, bz_ref, bh_ref)
    st0p = st0_ref[0]
    wa = wa_ref[...]
    wo = wo_ref[...]
    acc = jnp.zeros((_NP,), jnp.float32)
    for h in range(2):
        st = ns[:, h * _D:(h + 1) * _D]
        ann = st0p[:, h * _D:h * _D + _ANN]
        la = (jnp.sum(st * wa[0, :_D][None, :], axis=1)
              + jnp.sum(ann * wa[0, _D:][None, :], axis=1) + ba_ref[0, 0])
        lo = (jnp.sum(st * wo[0, :_D][None, :], axis=1)
              + jnp.sum(ann * wo[0, _D:][None, :], axis=1) + bo_ref[0, 0])
        acc = acc + jax.nn.sigmoid(la) * jnp.tanh(lo)
    out_ref[0] = jnp.broadcast_to(jax.nn.sigmoid(jnp.sum(acc)), (8, 128))


def _gru_readout(acat2, state3, W_r, W_z, W_h, brP, bzP, bhP,
                 state0, W_a, b_a, W_o, b_o):
    aspec_in = pl.BlockSpec((1, 1, _NP, _PD), lambda b: (1, b, 0, 0))
    aspec_out = pl.BlockSpec((1, 1, _NP, _PD), lambda b: (0, b, 0, 0))
    full3 = pl.BlockSpec((1, _NP, _PD), lambda b: (b, 0, 0))
    wspec = pl.BlockSpec((3 * _D, _D), lambda b: (0, 0))
    bspec = pl.BlockSpec((1, _PD), lambda b: (0, 0))
    sspec = pl.BlockSpec((1, 1), lambda b: (0, 0))
    return pl.pallas_call(
        _gru_read_body,
        grid=(_B,),
        in_specs=[aspec_in, aspec_out, full3, wspec, wspec, wspec,
                  bspec, bspec, bspec, full3,
                  pl.BlockSpec((1, _D + _ANN), lambda b: (0, 0)),
                  pl.BlockSpec((1, _D + _ANN), lambda b: (0, 0)),
                  sspec, sspec],
        out_specs=pl.BlockSpec((1, 8, 128), lambda b: (b, 0, 0)),
        out_shape=jax.ShapeDtypeStruct((_B, 8, 128), jnp.float32),
    )(acat2, acat2, state3, W_r, W_z, W_h, brP, bzP, bhP, state0,
      W_a.reshape(1, _D + _ANN), W_o.reshape(1, _D + _ANN),
      b_a.reshape(1, 1), b_o.reshape(1, 1))


def kernel(annotation_id, A, edgeEmbed, edgeBias, typeEmbed,
           W_r, b_r, W_z, b_z, W_h, b_h, W_a, b_a, W_o, b_o):
    gidx, didx = _edge_indices(A)
    M3 = edgeEmbed.reshape(_NET, _D, _D)
    biasP = jnp.concatenate([edgeBias, edgeBias], axis=1).reshape(_NET, 1, _PD)

    def pair(v):
        return jnp.concatenate([v, v]).reshape(1, _PD)

    brP, bzP, bhP = pair(b_r), pair(b_z), pair(b_h)
    sc_pass = _make_sc_edge_pass()

    state0, tr = _init_transform(annotation_id, typeEmbed, M3, biasP)
    acat2 = sc_pass(tr.reshape(_NET * _B * _N, _D),
                    gidx, didx).reshape(2, _B, _NP, _PD)
    state1, tr = _gru_transform(acat2, state0, W_r, W_z, W_h,
                                brP, bzP, bhP, M3, biasP)
    acat2 = sc_pass(tr.reshape(_NET * _B * _N, _D),
                    gidx, didx).reshape(2, _B, _NP, _PD)
    return _gru_readout(acat2, state1, W_r, W_z, W_h, brP, bzP, bhP,
                        state0, W_a, b_a, W_o, b_o)[:, 0, 0]


# speedup vs baseline: 1.4932x; 1.4932x over previous
import functools

import jax
import jax.numpy as jnp
from jax import lax
from jax.experimental import pallas as pl
from jax.experimental.pallas import tpu as pltpu
from jax.experimental.pallas import tpu_sc as plsc

_B = 4
_N = 4096
_DEG = 16
_D = 64
_ANN = 32
_NT = 32
_NET = 16
_STEPS = 2

_NP = _N // 2
_PD = 2 * _D

_NC = 2
_NS = 16
_LANES = 128
_E = _B * _N * _DEG
_EROWS = _E // _LANES
_CHUNKS = 2 * _E // (_NC * _NS * _LANES)
_ACC_ROWS = 2 * 2 * _N
_WB = _ACC_ROWS // _NS


def _blockdiag(m):
    z = jnp.zeros((_D, _D), jnp.float32)
    return jnp.concatenate([jnp.concatenate([m, z], axis=1),
                            jnp.concatenate([z, m], axis=1)], axis=0)


def _index_body(et_ref, nb_ref, gidx_ref, didx_ref):
    et = et_ref[...]
    nb = nb_ref[...]
    f = (lax.broadcasted_iota(jnp.int32, (_EROWS, _LANES), 0) * _LANES
         + lax.broadcasted_iota(jnp.int32, (_EROWS, _LANES), 1))
    b = f // (_N * _DEG)
    n = (f // _DEG) % _N
    bloc = (b % 2) * _N
    g_out = (et - 1) * (_B * _N) + b * _N + nb
    g_in = (et + _NET // 2 - 1) * (_B * _N) + b * _N + n
    d_out = bloc + n
    d_in = 2 * _N + bloc + nb
    half = _EROWS // _NC
    for c in range(_NC):
        sl = slice(c * half, (c + 1) * half)
        gidx_ref[c, 0:half] = g_out[sl]
        gidx_ref[c, half:2 * half] = g_in[sl]
        didx_ref[c, 0:half] = d_out[sl]
        didx_ref[c, half:2 * half] = d_in[sl]


def _edge_indices(A):
    et = A[..., 0].reshape(_EROWS, _LANES)
    nb = A[..., 1].reshape(_EROWS, _LANES)
    sh = jax.ShapeDtypeStruct((_NC, 2 * _EROWS // _NC, _LANES), jnp.int32)
    return pl.pallas_call(_index_body, out_shape=(sh, sh))(et, nb)


def _emit_transform(x, m3_ref, biasP_ref, tr_ref):
    for t in range(_NET):
        mb = _blockdiag(m3_ref[t])
        y = lax.dot_general(x, mb, (((1,), (1,)), ((), ())),
                            preferred_element_type=jnp.float32)
        tr_ref[t, 0] = y + biasP_ref[t]


def _gru_value(aip, aop, stp, wr_ref, wz_ref, wh_ref, br_ref, bz_ref, bh_ref):
    def mm3(x0, x1, x2, w_ref):
        w = w_ref[...]
        acc = jnp.zeros((_NP, _PD), jnp.float32)
        for k, x in enumerate((x0, x1, x2)):
            wb = _blockdiag(w[k * _D:(k + 1) * _D])
            acc = acc + jnp.dot(x, wb, preferred_element_type=jnp.float32)
        return acc

    r = jax.nn.sigmoid(mm3(aip, aop, stp, wr_ref) + br_ref[...])
    z = jax.nn.sigmoid(mm3(aip, aop, stp, wz_ref) + bz_ref[...])
    hh = jnp.tanh(mm3(aip, aop, r * stp, wh_ref) + bh_ref[...])
    return (1.0 - z) * stp + z * hh


def _init_tr_body(ids_ref, te_ref, m3_ref, biasP_ref, st_ref, tr_ref):
    ids = ids_ref[0]
    te = te_ref[...]
    iota = lax.broadcasted_iota(jnp.int32, (_NP, _NT), 1)
    halves = []
    zpad = jnp.zeros((_NP, _D - _ANN), jnp.float32)
    for h in range(2):
        idh = ids[:, h:h + 1]
        oh = jnp.where((idh - 1 == iota) & (idh > 0), 1.0, 0.0)
        halves.append(jnp.dot(oh, te, preferred_element_type=jnp.float32))
        halves.append(zpad)
    x = jnp.concatenate(halves, axis=1)
    st_ref[0] = x
    _emit_transform(x, m3_ref, biasP_ref, tr_ref)


def _init_transform(annotation_id, typeEmbed, M3, biasP):
    return pl.pallas_call(
        _init_tr_body,
        grid=(_B,),
        in_specs=[
            pl.BlockSpec((1, _NP, 2), lambda b: (b, 0, 0)),
            pl.BlockSpec((_NT, _ANN), lambda b: (0, 0)),
            pl.BlockSpec((_NET, _D, _D), lambda b: (0, 0, 0)),
            pl.BlockSpec((_NET, 1, _PD), lambda b: (0, 0, 0)),
        ],
        out_specs=(
            pl.BlockSpec((1, _NP, _PD), lambda b: (b, 0, 0)),
            pl.BlockSpec((_NET, 1, _NP, _PD), lambda b: (0, b, 0, 0)),
        ),
        out_shape=(
            jax.ShapeDtypeStruct((_B, _NP, _PD), jnp.float32),
            jax.ShapeDtypeStruct((_NET, _B, _NP, _PD), jnp.float32),
        ),
    )(annotation_id.reshape(_B, _NP, 2), typeEmbed, M3, biasP)


_NBUF = 4


@functools.cache
def _make_sc_edge_pass():
    mesh = plsc.VectorSubcoreMesh(core_axis_name="c", subcore_axis_name="s")

    @functools.partial(
        pl.kernel,
        out_type=jax.ShapeDtypeStruct((2 * _B * _N, _D), jnp.float32),
        mesh=mesh,
        scratch_types=[
            pltpu.VMEM((_CHUNKS, _LANES), jnp.int32),
            pltpu.VMEM((_CHUNKS, _LANES), jnp.int32),
            pltpu.VMEM((_NBUF, _LANES, _D), jnp.float32),
            pltpu.VMEM_SHARED((_ACC_ROWS, _D), jnp.float32),
        ] + [pltpu.SemaphoreType.DMA] * _NBUF,
        compiler_params=pltpu.CompilerParams(use_tc_tiling_on_sc=False),
    )
    def _sc_edge_pass(tr_hbm, gidx_hbm, didx_hbm, out_hbm,
                      gidx_v, didx_v, rows_v, acc, *sems):
        c = lax.axis_index("c")
        s = lax.axis_index("s")
        pltpu.sync_copy(gidx_hbm.at[c, pl.ds(s * _CHUNKS, _CHUNKS)], gidx_v)
        pltpu.sync_copy(didx_hbm.at[c, pl.ds(s * _CHUNKS, _CHUNKS)], didx_v)

        def _zero_row(i, carry):
            for j in range(_D // 16):
                rows_v[0, i, pl.ds(j * 16, 16)] = jnp.zeros((16,), jnp.float32)
            return carry

        lax.fori_loop(0, _LANES, _zero_row, 0)
        for k in range(_WB // _LANES):
            pltpu.sync_copy(rows_v.at[0],
                            acc.at[pl.ds(s * _WB + k * _LANES, _LANES)])
        plsc.subcore_barrier()

        def _gather_start(i, b):
            pltpu.async_copy(tr_hbm.at[gidx_v.at[i]], rows_v.at[b], sems[b])

        def _gather_wait(i, b):
            pltpu.make_async_copy(tr_hbm.at[gidx_v.at[i]], rows_v.at[b],
                                  sems[b]).wait()

        def _scatter(i, b):
            pltpu.sync_copy(rows_v.at[b], acc.at[didx_v.at[i]], add=True)

        for b in range(_NBUF):
            _gather_start(b, b)

        def _quad(k, carry):
            i = _NBUF * k
            for b in range(_NBUF):
                _gather_wait(i + b, b)
                _scatter(i + b, b)
                _gather_start(i + _NBUF + b, b)
            return carry

        lax.fori_loop(0, _CHUNKS // _NBUF - 1, _quad, 0)
        i = _CHUNKS - _NBUF
        for b in range(_NBUF):
            _gather_wait(i + b, b)
            _scatter(i + b, b)
        plsc.subcore_barrier()

        base = s * _WB + (s // (_NS // 2)) * (2 * _N) + c * (2 * _N)
        pltpu.sync_copy(acc.at[pl.ds(s * _WB, _WB)], out_hbm.at[pl.ds(base, _WB)])

    return _sc_edge_pass


def _gru_tr_body(ai_ref, ao_ref, st_ref, wr_ref, wz_ref, wh_ref,
                 br_ref, bz_ref, bh_ref, m3_ref, biasP_ref, out_ref, tr_ref):
    ns = _gru_value(ai_ref[0, 0], ao_ref[0, 0], st_ref[0],
                    wr_ref, wz_ref, wh_ref, br_ref, bz_ref, bh_ref)
    out_ref[0] = ns
    _emit_transform(ns, m3_ref, biasP_ref, tr_ref)


def _gru_transform(acat2, state3, W_r, W_z, W_h, brP, bzP, bhP, M3, biasP):
    aspec_in = pl.BlockSpec((1, 1, _NP, _PD), lambda b: (1, b, 0, 0))
    aspec_out = pl.BlockSpec((1, 1, _NP, _PD), lambda b: (0, b, 0, 0))
    full3 = pl.BlockSpec((1, _NP, _PD), lambda b: (b, 0, 0))
    wspec = pl.BlockSpec((3 * _D, _D), lambda b: (0, 0))
    bspec = pl.BlockSpec((1, _PD), lambda b: (0, 0))
    return pl.pallas_call(
        _gru_tr_body,
        grid=(_B,),
        in_specs=[aspec_in, aspec_out, full3, wspec, wspec, wspec,
                  bspec, bspec, bspec,
                  pl.BlockSpec((_NET, _D, _D), lambda b: (0, 0, 0)),
                  pl.BlockSpec((_NET, 1, _PD), lambda b: (0, 0, 0))],
        out_specs=(
            full3,
            pl.BlockSpec((_NET, 1, _NP, _PD), lambda b: (0, b, 0, 0)),
        ),
        out_shape=(
            jax.ShapeDtypeStruct((_B, _NP, _PD), jnp.float32),
            jax.ShapeDtypeStruct((_NET, _B, _NP, _PD), jnp.float32),
        ),
    )(acat2, acat2, state3, W_r, W_z, W_h, brP, bzP, bhP, M3, biasP)


def _gru_read_body(ai_ref, ao_ref, st_ref, wr_ref, wz_ref, wh_ref,
                   br_ref, bz_ref, bh_ref, st0_ref, wa_ref, wo_ref,
                   ba_ref, bo_ref, out_ref):
    ns = _gru_value(ai_ref[0, 0], ao_ref[0, 0], st_ref[0],
                    wr_ref, wz_ref, wh_ref, br_ref, bz_ref, bh_ref)
    st0p = st0_ref[0]
    wa = wa_ref[...]
    wo = wo_ref[...]
    acc = jnp.zeros((_NP,), jnp.float32)
    for h in range(2):
        st = ns[:, h * _D:(h + 1) * _D]
        ann = st0p[:, h * _D:h * _D + _ANN]
        la = (jnp.sum(st * wa[0, :_D][None, :], axis=1)
              + jnp.sum(ann * wa[0, _D:][None, :], axis=1) + ba_ref[0, 0])
        lo = (jnp.sum(st * wo[0, :_D][None, :], axis=1)
              + jnp.sum(ann * wo[0, _D:][None, :], axis=1) + bo_ref[0, 0])
        acc = acc + jax.nn.sigmoid(la) * jnp.tanh(lo)
    out_ref[0] = jnp.broadcast_to(jax.nn.sigmoid(jnp.sum(acc)), (8, 128))


def _gru_readout(acat2, state3, W_r, W_z, W_h, brP, bzP, bhP,
                 state0, W_a, b_a, W_o, b_o):
    aspec_in = pl.BlockSpec((1, 1, _NP, _PD), lambda b: (1, b, 0, 0))
    aspec_out = pl.BlockSpec((1, 1, _NP, _PD), lambda b: (0, b, 0, 0))
    full3 = pl.BlockSpec((1, _NP, _PD), lambda b: (b, 0, 0))
    wspec = pl.BlockSpec((3 * _D, _D), lambda b: (0, 0))
    bspec = pl.BlockSpec((1, _PD), lambda b: (0, 0))
    sspec = pl.BlockSpec((1, 1), lambda b: (0, 0))
    return pl.pallas_call(
        _gru_read_body,
        grid=(_B,),
        in_specs=[aspec_in, aspec_out, full3, wspec, wspec, wspec,
                  bspec, bspec, bspec, full3,
                  pl.BlockSpec((1, _D + _ANN), lambda b: (0, 0)),
                  pl.BlockSpec((1, _D + _ANN), lambda b: (0, 0)),
                  sspec, sspec],
        out_specs=pl.BlockSpec((1, 8, 128), lambda b: (b, 0, 0)),
        out_shape=jax.ShapeDtypeStruct((_B, 8, 128), jnp.float32),
    )(acat2, acat2, state3, W_r, W_z, W_h, brP, bzP, bhP, state0,
      W_a.reshape(1, _D + _ANN), W_o.reshape(1, _D + _ANN),
      b_a.reshape(1, 1), b_o.reshape(1, 1))


def kernel(annotation_id, A, edgeEmbed, edgeBias, typeEmbed,
           W_r, b_r, W_z, b_z, W_h, b_h, W_a, b_a, W_o, b_o):
    gidx, didx = _edge_indices(A)
    M3 = edgeEmbed.reshape(_NET, _D, _D)
    biasP = jnp.concatenate([edgeBias, edgeBias], axis=1).reshape(_NET, 1, _PD)

    def pair(v):
        return jnp.concatenate([v, v]).reshape(1, _PD)

    brP, bzP, bhP = pair(b_r), pair(b_z), pair(b_h)
    sc_pass = _make_sc_edge_pass()

    state0, tr = _init_transform(annotation_id, typeEmbed, M3, biasP)
    acat2 = sc_pass(tr.reshape(_NET * _B * _N, _D),
                    gidx, didx).reshape(2, _B, _NP, _PD)
    state1, tr = _gru_transform(acat2, state0, W_r, W_z, W_h,
                                brP, bzP, bhP, M3, biasP)
    acat2 = sc_pass(tr.reshape(_NET * _B * _N, _D),
                    gidx, didx).reshape(2, _B, _NP, _PD)
    return _gru_readout(acat2, state1, W_r, W_z, W_h, brP, bzP, bhP,
                        state0, W_a, b_a, W_o, b_o)[:, 0, 0]
